# baseline (speedup 1.0000x reference)
import dataclasses
import functools

import jax
import jax.numpy as jnp
from jax import lax
from jax.experimental import pallas as pl
from jax.experimental.pallas import tpu as pltpu
from jax.experimental.pallas import tpu_sc as plsc

B = 16384
F_SPARSE = 26
VOCAB = 100000
F_DENSE = 13

NUM_CORES = 2
NUM_SUBCORES = 16
NW = NUM_CORES * NUM_SUBCORES
BW = B // NW
CHUNKS = BW // 16
KPF = BW // 128
IDX_ROWS = F_SPARSE * KPF

_CP = pltpu.CompilerParams()
if "needs_layout_passes" in pltpu.CompilerParams.__dataclass_fields__:
    _CP = dataclasses.replace(_CP, needs_layout_passes=False)


@functools.partial(
    pl.kernel,
    out_type=jax.ShapeDtypeStruct((B,), jnp.float32),
    mesh=plsc.VectorSubcoreMesh(core_axis_name="c", subcore_axis_name="s"),
    compiler_params=_CP,
    scratch_types=[
        pltpu.VMEM((IDX_ROWS, 128), jnp.int32),
        pltpu.VMEM((IDX_ROWS, 128), jnp.float32),
        pltpu.VMEM((F_DENSE, BW), jnp.float32),
        pltpu.VMEM((F_DENSE, 16), jnp.float32),
        pltpu.VMEM((BW,), jnp.float32),
        pltpu.SemaphoreType.DMA,
        pltpu.SemaphoreType.DMA,
        pltpu.SemaphoreType.DMA,
        pltpu.SemaphoreType.DMA,
    ],
)
def _linear_logit_sc(xs_hbm, xd_hbm, w_hbm, *rest):
    tabs, (out_hbm, idx_v, g_v, xd_v, w_v, out_v,
           sem_i, sem_x, sem_w, sem_g) = rest[:F_SPARSE], rest[F_SPARSE:]
    wid = lax.axis_index("s") * NUM_CORES + lax.axis_index("c")
    base = wid * BW
    idx_cps = [
        pltpu.async_copy(
            xs_hbm.at[f, pl.ds(base + 128 * k, 128)],
            idx_v.at[f * KPF + k], sem_i)
        for f in range(F_SPARSE) for k in range(KPF)
    ]
    cp_xd = pltpu.async_copy(xd_hbm.at[:, pl.ds(base, BW)], xd_v, sem_x)
    cp_w = pltpu.async_copy(w_hbm, w_v, sem_w)
    for cp in idx_cps:
        cp.wait()
    g_cps = [
        pltpu.async_copy(tabs[f].at[idx_v.at[f * KPF + k]],
                         g_v.at[f * KPF + k], sem_g)
        for f in range(F_SPARSE) for k in range(KPF)
    ]
    cp_xd.wait()
    cp_w.wait()
    half = F_SPARSE // 2
    for cp in g_cps[:half * KPF]:
        cp.wait()
    wvecs = [w_v[d] for d in range(F_DENSE)]
    for c in range(CHUNKS):
        k, off = c // 8, (c % 8) * 16
        sl = pl.ds(off, 16)
        acc = g_v[k, sl]
        for f in range(1, half):
            acc = acc + g_v[f * KPF + k, sl]
        csl = pl.ds(c * 16, 16)
        for d in range(F_DENSE):
            acc = acc + xd_v[d, csl] * wvecs[d]
        out_v[csl] = acc
    for cp in g_cps[half * KPF:]:
        cp.wait()
    for c in range(CHUNKS):
        k, off = c // 8, (c % 8) * 16
        sl = pl.ds(off, 16)
        csl = pl.ds(c * 16, 16)
        acc = out_v[csl]
        for f in range(half, F_SPARSE):
            acc = acc + g_v[f * KPF + k, sl]
        out_v[csl] = acc
    pltpu.sync_copy(out_v, out_hbm.at[pl.ds(base, BW)])


def kernel(X_sparse, X_dense, tables, W):
    xs_t = X_sparse.astype(jnp.int32).T
    xd_t = X_dense.T
    wsp = jnp.broadcast_to(W, (F_DENSE, 16))
    tabs = [lax.optimization_barrier(tables[f, :, 0])
            for f in range(F_SPARSE)]
    out = _linear_logit_sc(xs_t, xd_t, wsp, *tabs)
    return out.reshape(B, 1)


# speedup vs baseline: 1.0044x; 1.0044x over previous
import dataclasses
import functools

import jax
import jax.numpy as jnp
from jax import lax
from jax.experimental import pallas as pl
from jax.experimental.pallas import tpu as pltpu
from jax.experimental.pallas import tpu_sc as plsc

B = 16384
F_SPARSE = 26
VOCAB = 100000
F_DENSE = 13

NUM_CORES = 2
NUM_SUBCORES = 16
NW = NUM_CORES * NUM_SUBCORES
BW = B // NW
CHUNKS = BW // 16
KPF = BW // 128
IDX_ROWS = F_SPARSE * KPF

_CP = pltpu.CompilerParams()
if "needs_layout_passes" in pltpu.CompilerParams.__dataclass_fields__:
    _CP = dataclasses.replace(_CP, needs_layout_passes=False)


@functools.partial(
    pl.kernel,
    out_type=jax.ShapeDtypeStruct((B,), jnp.float32),
    mesh=plsc.VectorSubcoreMesh(core_axis_name="c", subcore_axis_name="s"),
    compiler_params=_CP,
    scratch_types=[
        pltpu.VMEM((IDX_ROWS, 128), jnp.int32),
        pltpu.VMEM((IDX_ROWS, 128), jnp.float32),
        pltpu.VMEM((F_DENSE, BW), jnp.float32),
        pltpu.VMEM((F_DENSE, 16), jnp.float32),
        pltpu.VMEM((BW,), jnp.float32),
        pltpu.SemaphoreType.DMA,
        pltpu.SemaphoreType.DMA,
        pltpu.SemaphoreType.DMA,
        pltpu.SemaphoreType.DMA,
    ],
)
def _linear_logit_sc(xs_hbm, xd_hbm, w_hbm, *rest):
    tabs, (out_hbm, idx_v, g_v, xd_v, w_v, out_v,
           sem_i, sem_x, sem_w, sem_g) = rest[:F_SPARSE], rest[F_SPARSE:]
    wid = lax.axis_index("s") * NUM_CORES + lax.axis_index("c")
    base = wid * BW
    idx_cps = [
        pltpu.async_copy(
            xs_hbm.at[f, pl.ds(base + 128 * k, 128)],
            idx_v.at[f * KPF + k], sem_i)
        for f in range(F_SPARSE) for k in range(KPF)
    ]
    cp_xd = pltpu.async_copy(xd_hbm.at[:, pl.ds(base, BW)], xd_v, sem_x)
    cp_w = pltpu.async_copy(w_hbm, w_v, sem_w)
    for cp in idx_cps:
        cp.wait()
    g_cps = [
        pltpu.async_copy(tabs[f].at[idx_v.at[f * KPF + k]],
                         g_v.at[f * KPF + k], sem_g)
        for f in range(F_SPARSE) for k in range(KPF)
    ]
    cp_xd.wait()
    cp_w.wait()
    for cp in g_cps:
        cp.wait()

    wvecs = [w_v[d] for d in range(F_DENSE)]
    for c in range(CHUNKS):
        k, off = c // 8, (c % 8) * 16
        sl = pl.ds(off, 16)
        acc = g_v[k, sl]
        for f in range(1, F_SPARSE):
            acc = acc + g_v[f * KPF + k, sl]
        csl = pl.ds(c * 16, 16)
        for d in range(F_DENSE):
            acc = acc + xd_v[d, csl] * wvecs[d]
        out_v[csl] = acc
    pltpu.sync_copy(out_v, out_hbm.at[pl.ds(base, BW)])


def kernel(X_sparse, X_dense, tables, W):
    xs_t = X_sparse.astype(jnp.int32).T
    xd_t = X_dense.T
    wsp = jnp.broadcast_to(W, (F_DENSE, 16))
    tabs = [tables[f, :, 0] for f in range(F_SPARSE)]
    out = _linear_logit_sc(xs_t, xd_t, wsp, *tabs)
    return out.reshape(B, 1)
